# baseline (speedup 1.0000x reference)
import jax
import jax.numpy as jnp
from jax.experimental import pallas as pl
from jax.experimental.pallas import tpu as pltpu

B, N, D_IN, D_OUT = 2, 2048, 1024, 1024
BM = 1024


def _proj_kernel(x_ref, w_ref, o_ref):
    xp = jax.lax.dot_general(
        x_ref[...], w_ref[...], (((1,), (1,)), ((), ())),
        preferred_element_type=jnp.float32)
    o_ref[...] = xp.astype(jnp.bfloat16)


def _attn_kernel(xp_blk_ref, xp_all_ref, att_ref, h_ref):
    xp_blk = xp_blk_ref[...]
    xp_all = xp_all_ref[...]
    score = jax.lax.dot_general(
        xp_blk, xp_all, (((1,), (1,)), ((), ())),
        preferred_element_type=jnp.float32)
    rowmin = jnp.min(score, axis=1)
    att = jnp.where(score == rowmin[:, None], jnp.float32(1.0), jnp.float32(0.0))
    att_ref[...] = att
    att_bf = att.astype(jnp.bfloat16)
    h_ref[...] = jax.lax.dot_general(
        att_bf, xp_all, (((1,), (0,)), ((), ())),
        preferred_element_type=jnp.float32)


def kernel(x, W):
    nb = N // BM
    x = x.astype(jnp.bfloat16)
    W = W.astype(jnp.bfloat16)
    x_proj = pl.pallas_call(
        _proj_kernel,
        grid=(B, nb),
        in_specs=[
            pl.BlockSpec((None, BM, D_IN), lambda b, i: (b, i, 0)),
            pl.BlockSpec((D_OUT, D_IN), lambda b, i: (0, 0)),
        ],
        out_specs=pl.BlockSpec((None, BM, D_OUT), lambda b, i: (b, i, 0)),
        out_shape=jax.ShapeDtypeStruct((B, N, D_OUT), jnp.bfloat16),
        compiler_params=pltpu.CompilerParams(
            dimension_semantics=("parallel", "parallel")),
    )(x, W)

    att, h = pl.pallas_call(
        _attn_kernel,
        grid=(B, nb),
        in_specs=[
            pl.BlockSpec((None, BM, D_OUT), lambda b, i: (b, i, 0)),
            pl.BlockSpec((None, N, D_OUT), lambda b, i: (b, 0, 0)),
        ],
        out_specs=[
            pl.BlockSpec((None, BM, N), lambda b, i: (b, i, 0)),
            pl.BlockSpec((None, BM, D_OUT), lambda b, i: (b, i, 0)),
        ],
        out_shape=[
            jax.ShapeDtypeStruct((B, N, N), jnp.float32),
            jax.ShapeDtypeStruct((B, N, D_OUT), jnp.float32),
        ],
        compiler_params=pltpu.CompilerParams(
            dimension_semantics=("parallel", "parallel")),
    )(x_proj, x_proj)
    return (h, att)


# speedup vs baseline: 1.1986x; 1.1986x over previous
import jax
import jax.numpy as jnp
from jax.experimental import pallas as pl
from jax.experimental.pallas import tpu as pltpu

B, N, D_IN, D_OUT = 2, 2048, 1024, 1024
BM = 1024


def _proj_kernel(x_ref, w_ref, o_ref):
    xp = jax.lax.dot_general(
        x_ref[...], w_ref[...], (((1,), (1,)), ((), ())),
        preferred_element_type=jnp.float32)
    o_ref[...] = xp.astype(jnp.bfloat16)


def _attn_kernel(xp_blk_ref, xp_all_ref, att_ref, h_ref):
    xp_blk = xp_blk_ref[...]
    xp_all = xp_all_ref[...]
    score = jax.lax.dot_general(
        xp_blk, xp_all, (((1,), (1,)), ((), ())),
        preferred_element_type=jnp.float32)
    rowmin = jnp.min(score, axis=1)
    att = jnp.where(score == rowmin[:, None], jnp.float32(1.0), jnp.float32(0.0))
    att_ref[...] = att
    att_bf = att.astype(jnp.bfloat16)
    h_ref[...] = jax.lax.dot_general(
        att_bf, xp_all, (((1,), (0,)), ((), ())),
        preferred_element_type=jnp.float32)


def kernel(x, W):
    nb = N // BM
    x_proj = pl.pallas_call(
        _proj_kernel,
        grid=(B, nb),
        in_specs=[
            pl.BlockSpec((None, BM, D_IN), lambda b, i: (b, i, 0)),
            pl.BlockSpec((D_OUT, D_IN), lambda b, i: (0, 0)),
        ],
        out_specs=pl.BlockSpec((None, BM, D_OUT), lambda b, i: (b, i, 0)),
        out_shape=jax.ShapeDtypeStruct((B, N, D_OUT), jnp.bfloat16),
        compiler_params=pltpu.CompilerParams(
            dimension_semantics=("parallel", "parallel")),
    )(x, W)

    att, h = pl.pallas_call(
        _attn_kernel,
        grid=(B, nb),
        in_specs=[
            pl.BlockSpec((None, BM, D_OUT), lambda b, i: (b, i, 0)),
            pl.BlockSpec((None, N, D_OUT), lambda b, i: (b, 0, 0)),
        ],
        out_specs=[
            pl.BlockSpec((None, BM, N), lambda b, i: (b, i, 0)),
            pl.BlockSpec((None, BM, D_OUT), lambda b, i: (b, i, 0)),
        ],
        out_shape=[
            jax.ShapeDtypeStruct((B, N, N), jnp.float32),
            jax.ShapeDtypeStruct((B, N, D_OUT), jnp.float32),
        ],
        compiler_params=pltpu.CompilerParams(
            dimension_semantics=("parallel", "parallel")),
    )(x_proj, x_proj)
    return (h, att)
